# baseline (speedup 1.0000x reference)
import functools

import jax
import jax.numpy as jnp
from jax import lax
from jax.experimental import pallas as pl
from jax.experimental.pallas import tpu as pltpu
from jax.experimental.pallas import tpu_sc as plsc

_NC = 2
_NS = 16
_NW = _NC * _NS
_L = 16

_CUT = 5.0


def _mlp_body(sf_ref, w1_ref, b1_ref, w2_ref, b2_ref, out_ref):
    h = jnp.dot(sf_ref[...], w1_ref[...], preferred_element_type=jnp.float32)
    h = h + b1_ref[...]
    h = (h * jax.nn.sigmoid(h)) * (1.0 / 0.6)
    out_ref[...] = (
        jnp.dot(h, w2_ref[...], preferred_element_type=jnp.float32) + b2_ref[...]
    ).astype(jnp.bfloat16)


def _node_mlp(sf, w1, b1, w2, b2):
    n, d = sf.shape
    d3 = w2.shape[1]
    blk = 1000
    grid = n // blk
    return pl.pallas_call(
        _mlp_body,
        grid=(grid,),
        in_specs=[
            pl.BlockSpec((blk, d), lambda i: (i, 0)),
            pl.BlockSpec((d, d), lambda i: (0, 0)),
            pl.BlockSpec((1, d), lambda i: (0, 0)),
            pl.BlockSpec((d, d3), lambda i: (0, 0)),
            pl.BlockSpec((1, d3), lambda i: (0, 0)),
        ],
        out_specs=pl.BlockSpec((blk, d3), lambda i: (i, 0)),
        out_shape=jax.ShapeDtypeStruct((n, d3), jnp.bfloat16),
    )(sf, w1, b1, w2, b2)


_GA_BLK = 80


def _gather_body(s_h, vf_h, px_h, py_h, pz_h, src_h, tgt_h,
                 s_out, vf_out, ex_h, ey_h, ez_h, ed2_h, *scratch):
    set0 = scratch[0:12]
    set1 = scratch[12:24]
    bx, by, bz, bd = scratch[24:28]
    n_edges = src_h.shape[0]
    per_tile = n_edges // _NW
    nblk = per_tile // _GA_BLK
    c = lax.axis_index("c")
    s = lax.axis_index("s")
    wid = s * _NC + c
    base0 = wid * per_tile

    def fire_idx(b, st):
        src_v, tgt_v = st[0], st[1]
        sem_i = st[10]
        sl_e = pl.ds(base0 + b * _GA_BLK, _GA_BLK)
        pltpu.async_copy(src_h.at[sl_e], src_v, sem_i)
        pltpu.async_copy(tgt_h.at[sl_e], tgt_v, sem_i)

    def wait_idx(b, st):
        src_v, tgt_v = st[0], st[1]
        sem_i = st[10]
        sl_e = pl.ds(base0 + b * _GA_BLK, _GA_BLK)
        pltpu.make_async_copy(src_h.at[sl_e], src_v, sem_i).wait()
        pltpu.make_async_copy(tgt_h.at[sl_e], tgt_v, sem_i).wait()

    def _gather_list(st):
        src_v, tgt_v, rows_s, rows_v, gxs, gys, gzs, gxt, gyt, gzt = st[:10]
        sem_g = st[11]
        return [
            (s_h.at[src_v], rows_s, sem_g),
            (vf_h.at[src_v], rows_v, sem_g),
            (px_h.at[src_v], gxs, sem_g),
            (py_h.at[src_v], gys, sem_g),
            (pz_h.at[src_v], gzs, sem_g),
            (px_h.at[tgt_v], gxt, sem_g),
            (py_h.at[tgt_v], gyt, sem_g),
            (pz_h.at[tgt_v], gzt, sem_g),
        ]

    def fire_gathers(st):
        for args in _gather_list(st):
            pltpu.async_copy(*args)

    def wait_gathers(st):
        for args in _gather_list(st):
            pltpu.make_async_copy(*args).wait()

    def consume(b, st):
        rows_s, rows_v = st[2], st[3]
        gxs, gys, gzs, gxt, gyt, gzt = st[4:10]
        sl_e = pl.ds(base0 + b * _GA_BLK, _GA_BLK)

        def geom(i, carry2):
            sl = pl.ds(i * _L, _L)
            dx = gxt[sl] - gxs[sl]
            dy = gyt[sl] - gys[sl]
            dz = gzt[sl] - gzs[sl]
            bx[sl] = dx
            by[sl] = dy
            bz[sl] = dz
            bd[sl] = dx * dx + dy * dy + dz * dz
            return carry2

        lax.fori_loop(0, _GA_BLK // _L, geom, 0)
        pltpu.sync_copy(rows_s, s_out.at[sl_e, :])
        pltpu.sync_copy(rows_v, vf_out.at[sl_e, :])
        pltpu.sync_copy(bx, ex_h.at[sl_e])
        pltpu.sync_copy(by, ey_h.at[sl_e])
        pltpu.sync_copy(bz, ez_h.at[sl_e])
        pltpu.sync_copy(bd, ed2_h.at[sl_e])

    fire_idx(0, set0)
    wait_idx(0, set0)
    fire_gathers(set0)
    fire_idx(1, set1)

    def body(g, carry):
        b0 = g * 2
        b1 = b0 + 1
        wait_idx(b1, set1)
        fire_gathers(set1)
        wait_gathers(set0)

        @pl.when(b0 + 2 < nblk)
        def _i0():
            fire_idx(b0 + 2, set0)

        consume(b0, set0)

        @pl.when(b0 + 2 < nblk)
        def _g0():
            wait_idx(b0 + 2, set0)
            fire_gathers(set0)

        wait_gathers(set1)

        @pl.when(b1 + 2 < nblk)
        def _i1():
            fire_idx(b1 + 2, set1)

        consume(b1, set1)
        return carry

    lax.fori_loop(0, nblk // 2, body, 0)
    if nblk % 2:
        wait_gathers(set0)
        consume(nblk - 1, set0)


def _edge_gathers(s_packed, vf_packed, px, py, pz, src, tgt):
    n, dp = s_packed.shape
    e = src.shape[0]
    mesh = plsc.VectorSubcoreMesh(core_axis_name="c", subcore_axis_name="s")
    f32 = jnp.float32
    i32 = jnp.int32
    kfn = pl.kernel(
        _gather_body,
        mesh=mesh,
        out_type=(
            jax.ShapeDtypeStruct((e, dp), i32),
            jax.ShapeDtypeStruct((e, dp), i32),
            jax.ShapeDtypeStruct((e,), f32),
            jax.ShapeDtypeStruct((e,), f32),
            jax.ShapeDtypeStruct((e,), f32),
            jax.ShapeDtypeStruct((e,), f32),
        ),
        scratch_types=(
            [
                pltpu.VMEM((_GA_BLK,), jnp.int32),
                pltpu.VMEM((_GA_BLK,), jnp.int32),
                pltpu.VMEM((_GA_BLK, dp), i32),
                pltpu.VMEM((_GA_BLK, dp), i32),
            ]
            + [pltpu.VMEM((_GA_BLK,), f32) for _ in range(6)]
            + [pltpu.SemaphoreType.DMA, pltpu.SemaphoreType.DMA]
        ) * 2
        + [pltpu.VMEM((_GA_BLK,), f32) for _ in range(4)],
    )
    return kfn(s_packed, vf_packed, px, py, pz, src, tgt)


_TWO_PI = 6.2831853071795865
_COS_C = (
    0.9999994436787928,
    -0.49999558165605595,
    0.04166103279014615,
    -0.0013862747315839738,
    2.4253192495694853e-05,
    -2.2193949944515623e-07,
)


def _fast_cos(y):
    k = jnp.floor(y * (1.0 / _TWO_PI) + 0.5)
    t = y - k * _TWO_PI
    u = t * t
    r = jnp.float32(_COS_C[5])
    for c in _COS_C[4::-1]:
        r = r * u + jnp.float32(c)
    return r


def _unpack_bf16_pairs(w):
    lo = jax.lax.bitcast_convert_type(jax.lax.shift_left(w, 16), jnp.float32)
    hi = jax.lax.bitcast_convert_type(
        jax.lax.bitwise_and(w, jnp.int32(-65536)), jnp.float32
    )
    return jnp.concatenate([lo, hi], axis=1)


def _msg_body(ed_ref, ssrc_ref, vfsrc_ref, e32_ref, rw_ref, rb_ref, out_ref):
    d2 = ed_ref[:, 3:4]
    dist = jnp.sqrt(d2)
    inv_dist = 1.0 / dist
    ds_scaled = dist * (1.0 / _CUT)
    rbf = _fast_cos(ds_scaled * e32_ref[...] - (0.5 * jnp.pi)) * inv_dist
    filt = jnp.dot(rbf, rw_ref[...], preferred_element_type=jnp.float32)
    filt = filt + rb_ref[...]
    filt = jnp.where(
        filt < _CUT, 0.5 * (1.0 + _fast_cos(filt * (jnp.pi / _CUT))), 0.0
    )
    d = ssrc_ref.shape[1] // 2
    msg = _unpack_bf16_pairs(ssrc_ref[...])[:, :3 * d] * filt
    vfsrc = _unpack_bf16_pairs(vfsrc_ref[...])
    scalar = msg[:, :d]
    equiv = msg[:, d:2 * d]
    invar = msg[:, 2 * d:]
    out_ref[0] = scalar
    for k in range(3):
        ru_k = ed_ref[:, k:k + 1] * inv_dist
        out_ref[k + 1] = invar * ru_k + equiv * vfsrc[:, k * d:(k + 1) * d]


def _edge_messages(ed_t, s_src, vf_src, e32, rw, rb):
    e, dp = s_src.shape
    d = dp // 2
    d3 = 3 * d
    blk = 2560
    grid = e // blk
    return pl.pallas_call(
        _msg_body,
        grid=(grid,),
        in_specs=[
            pl.BlockSpec((blk, 4), lambda i: (i, 0)),
            pl.BlockSpec((blk, dp), lambda i: (i, 0)),
            pl.BlockSpec((blk, dp), lambda i: (i, 0)),
            pl.BlockSpec((1, e32.shape[1]), lambda i: (0, 0)),
            pl.BlockSpec((e32.shape[1], d3), lambda i: (0, 0)),
            pl.BlockSpec((1, d3), lambda i: (0, 0)),
        ],
        out_specs=pl.BlockSpec((4, blk, d), lambda i: (0, i, 0)),
        out_shape=jax.ShapeDtypeStruct((4, e, d), jnp.float32),
    )(ed_t, s_src, vf_src, e32, rw, rb)


_SC_BLK = 80


def _scatter_body(msg_h, tgt_h, zeros_h, out_h, acc,
                  idx0, buf0, sem0, idx1, buf1, sem1):
    n_edges = tgt_h.shape[0]
    n_nodes = zeros_h.shape[0]
    per_tile_e = n_edges // _NS
    rows_per_tile = (n_nodes // _NS) // 8 * 8
    rem = n_nodes - rows_per_tile * _NS
    nblk = per_tile_e // _SC_BLK
    c = lax.axis_index("c")
    s = lax.axis_index("s")
    r0 = s * rows_per_tile
    rrem = rows_per_tile * _NS
    ebase = s * per_tile_e

    for j in range(2):
        chunk = c * 2 + j
        pltpu.sync_copy(
            zeros_h.at[pl.ds(r0, rows_per_tile), :],
            acc.at[pl.ds(r0, rows_per_tile), :],
        )
        if rem:
            @pl.when(s == _NS - 1)
            def _zero_rem():
                pltpu.sync_copy(
                    zeros_h.at[pl.ds(rrem, rem), :],
                    acc.at[pl.ds(rrem, rem), :],
                )
        plsc.subcore_barrier()

        def start(b, idx_v, buf, sem):
            base = ebase + b * _SC_BLK
            pltpu.async_copy(tgt_h.at[pl.ds(base, _SC_BLK)], idx_v, sem)
            pltpu.async_copy(msg_h.at[chunk, pl.ds(base, _SC_BLK), :], buf, sem)

        def finish(b, idx_v, buf, sem):
            base = ebase + b * _SC_BLK
            pltpu.make_async_copy(
                tgt_h.at[pl.ds(base, _SC_BLK)], idx_v, sem
            ).wait()
            pltpu.make_async_copy(
                msg_h.at[chunk, pl.ds(base, _SC_BLK), :], buf, sem
            ).wait()
            pltpu.sync_copy(buf, acc.at[idx_v], add=True)

        start(0, idx0, buf0, sem0)

        def body(g, carry):
            b0 = g * 2
            start(b0 + 1, idx1, buf1, sem1)
            finish(b0, idx0, buf0, sem0)

            @pl.when(b0 + 2 < nblk)
            def _next():
                start(b0 + 2, idx0, buf0, sem0)

            finish(b0 + 1, idx1, buf1, sem1)
            return carry

        lax.fori_loop(0, nblk // 2, body, 0)
        plsc.subcore_barrier()
        pltpu.sync_copy(
            acc.at[pl.ds(r0, rows_per_tile), :],
            out_h.at[chunk, pl.ds(r0, rows_per_tile), :],
        )
        if rem:
            @pl.when(s == _NS - 1)
            def _drain_rem():
                pltpu.sync_copy(
                    acc.at[pl.ds(rrem, rem), :],
                    out_h.at[chunk, pl.ds(rrem, rem), :],
                )


def _scatter_add(msg4, tgt, zeros):
    e = tgt.shape[0]
    n, d = zeros.shape
    mesh = plsc.VectorSubcoreMesh(core_axis_name="c", subcore_axis_name="s")
    kfn = pl.kernel(
        _scatter_body,
        mesh=mesh,
        out_type=jax.ShapeDtypeStruct((4, n, d), jnp.float32),
        scratch_types=[
            pltpu.VMEM_SHARED((n, d), jnp.float32),
            pltpu.VMEM((_SC_BLK,), jnp.int32),
            pltpu.VMEM((_SC_BLK, d), jnp.float32),
            pltpu.SemaphoreType.DMA,
            pltpu.VMEM((_SC_BLK,), jnp.int32),
            pltpu.VMEM((_SC_BLK, d), jnp.float32),
            pltpu.SemaphoreType.DMA,
        ],
    )
    return kfn(msg4, tgt, zeros)


def kernel(vectorial_feat, scalar_feat, node_pos, edge_index, W1, b1, W2, b2,
           rbf_W, rbf_b, expanded_distance):
    n, d = scalar_feat.shape
    e = edge_index.shape[1]
    r = rbf_W.shape[0]
    d3 = 3 * d

    perm = (jnp.arange(d3) % d) * 3 + (jnp.arange(d3) // d)
    rwp = rbf_W[:, perm]
    rbp = rbf_b[perm].reshape(1, d3)
    d3e = 4 * d
    dp = d3e // 2
    k_idx = jnp.arange(d3e)
    f_perm = jnp.where(k_idx % 2 == 0, k_idx // 2, k_idx // 2 + dp)
    w2e = jnp.concatenate([W2[:, perm], jnp.zeros((d, d), jnp.float32)], axis=1)
    b2e = jnp.concatenate([b2[perm], jnp.zeros((d,), jnp.float32)])
    w2p = w2e[:, f_perm]
    b2p = b2e[f_perm].reshape(1, d3e)

    rp = 32
    rwp = jnp.concatenate([rwp, jnp.zeros((rp - r, d3), jnp.float32)], axis=0)
    e32 = jnp.concatenate(
        [expanded_distance, jnp.zeros((rp - r,), jnp.float32)]
    ).reshape(1, rp)

    vf_t = vectorial_feat.transpose(0, 2, 1).reshape(n, d3)
    vf_e = jnp.concatenate([vf_t, jnp.zeros((n, d), jnp.float32)], axis=1)
    vf_stored = vf_e[:, f_perm].astype(jnp.bfloat16)
    vf_packed = jax.lax.bitcast_convert_type(
        vf_stored.reshape(n, dp, 2), jnp.int32
    )
    src = edge_index[0]
    tgt = edge_index[1]
    px = node_pos[:, 0]
    py = node_pos[:, 1]
    pz = node_pos[:, 2]

    s = _node_mlp(scalar_feat, W1, b1.reshape(1, d), w2p, b2p)
    s_packed = jax.lax.bitcast_convert_type(s.reshape(n, dp, 2), jnp.int32)
    s_src, vf_src, ex, ey, ez, ed2 = _edge_gathers(
        s_packed, vf_packed, px, py, pz, src, tgt
    )
    ed_t = jnp.stack([ex, ey, ez, ed2], axis=1)
    msg4 = _edge_messages(ed_t, s_src, vf_src, e32, rwp, rbp)
    out4 = _scatter_add(msg4, tgt, jnp.zeros((n, d), jnp.float32))

    scalar_out = out4[0]
    vec_out = jnp.transpose(out4[1:], (1, 2, 0))
    return (vec_out, scalar_out)


# speedup vs baseline: 1.2127x; 1.2127x over previous
import functools

import jax
import jax.numpy as jnp
from jax import lax
from jax.experimental import pallas as pl
from jax.experimental.pallas import tpu as pltpu
from jax.experimental.pallas import tpu_sc as plsc

_NC = 2
_NS = 16
_NW = _NC * _NS
_L = 16

_CUT = 5.0


def _mlp_body(sf_ref, w1_ref, b1_ref, w2_ref, b2_ref, out_ref):
    h = jnp.dot(sf_ref[...], w1_ref[...], preferred_element_type=jnp.float32)
    h = h + b1_ref[...]
    h = (h * jax.nn.sigmoid(h)) * (1.0 / 0.6)
    out = jnp.dot(h, w2_ref[...], preferred_element_type=jnp.float32) + b2_ref[...]
    half = out.shape[1] // 2
    rnd = jnp.int32(0x8000)
    lo = jax.lax.bitcast_convert_type(out[:, :half], jnp.int32) + rnd
    hi = jax.lax.bitcast_convert_type(out[:, half:], jnp.int32) + rnd
    out_ref[...] = jax.lax.bitwise_or(
        jax.lax.shift_right_logical(lo, 16),
        jax.lax.bitwise_and(hi, jnp.int32(-65536)),
    )


def _node_mlp(sf, w1, b1, w2, b2):
    n, d = sf.shape
    d3 = w2.shape[1]
    blk = 1000
    grid = n // blk
    return pl.pallas_call(
        _mlp_body,
        grid=(grid,),
        in_specs=[
            pl.BlockSpec((blk, d), lambda i: (i, 0)),
            pl.BlockSpec((d, d), lambda i: (0, 0)),
            pl.BlockSpec((1, d), lambda i: (0, 0)),
            pl.BlockSpec((d, d3), lambda i: (0, 0)),
            pl.BlockSpec((1, d3), lambda i: (0, 0)),
        ],
        out_specs=pl.BlockSpec((blk, d3 // 2), lambda i: (i, 0)),
        out_shape=jax.ShapeDtypeStruct((n, d3 // 2), jnp.int32),
    )(sf, w1, b1, w2, b2)


_GA_BLK = 80


def _gather_body(s_h, vf_h, px_h, py_h, pz_h, src_h, tgt_h,
                 s_out, vf_out, ex_h, ey_h, ez_h, ed2_h, *scratch):
    set0 = scratch[0:12]
    set1 = scratch[12:24]
    bx, by, bz, bd = scratch[24:28]
    n_edges = src_h.shape[0]
    per_tile = n_edges // _NW
    nblk = per_tile // _GA_BLK
    c = lax.axis_index("c")
    s = lax.axis_index("s")
    wid = s * _NC + c
    base0 = wid * per_tile

    def fire_idx(b, st):
        src_v, tgt_v = st[0], st[1]
        sem_i = st[10]
        sl_e = pl.ds(base0 + b * _GA_BLK, _GA_BLK)
        pltpu.async_copy(src_h.at[sl_e], src_v, sem_i)
        pltpu.async_copy(tgt_h.at[sl_e], tgt_v, sem_i)

    def wait_idx(b, st):
        src_v, tgt_v = st[0], st[1]
        sem_i = st[10]
        sl_e = pl.ds(base0 + b * _GA_BLK, _GA_BLK)
        pltpu.make_async_copy(src_h.at[sl_e], src_v, sem_i).wait()
        pltpu.make_async_copy(tgt_h.at[sl_e], tgt_v, sem_i).wait()

    def _gather_list(st):
        src_v, tgt_v, rows_s, rows_v, gxs, gys, gzs, gxt, gyt, gzt = st[:10]
        sem_g = st[11]
        return [
            (s_h.at[src_v], rows_s, sem_g),
            (vf_h.at[src_v], rows_v, sem_g),
            (px_h.at[src_v], gxs, sem_g),
            (py_h.at[src_v], gys, sem_g),
            (pz_h.at[src_v], gzs, sem_g),
            (px_h.at[tgt_v], gxt, sem_g),
            (py_h.at[tgt_v], gyt, sem_g),
            (pz_h.at[tgt_v], gzt, sem_g),
        ]

    def fire_gathers(st):
        for args in _gather_list(st):
            pltpu.async_copy(*args)

    def wait_gathers(st):
        for args in _gather_list(st):
            pltpu.make_async_copy(*args).wait()

    def consume(b, st):
        rows_s, rows_v = st[2], st[3]
        gxs, gys, gzs, gxt, gyt, gzt = st[4:10]
        sl_e = pl.ds(base0 + b * _GA_BLK, _GA_BLK)

        def geom(i, carry2):
            sl = pl.ds(i * _L, _L)
            dx = gxt[sl] - gxs[sl]
            dy = gyt[sl] - gys[sl]
            dz = gzt[sl] - gzs[sl]
            bx[sl] = dx
            by[sl] = dy
            bz[sl] = dz
            bd[sl] = dx * dx + dy * dy + dz * dz
            return carry2

        lax.fori_loop(0, _GA_BLK // _L, geom, 0)
        pltpu.sync_copy(rows_s, s_out.at[sl_e, :])
        pltpu.sync_copy(rows_v, vf_out.at[sl_e, :])
        pltpu.sync_copy(bx, ex_h.at[sl_e])
        pltpu.sync_copy(by, ey_h.at[sl_e])
        pltpu.sync_copy(bz, ez_h.at[sl_e])
        pltpu.sync_copy(bd, ed2_h.at[sl_e])

    fire_idx(0, set0)
    wait_idx(0, set0)
    fire_gathers(set0)
    fire_idx(1, set1)

    def body(g, carry):
        b0 = g * 2
        b1 = b0 + 1
        wait_idx(b1, set1)
        fire_gathers(set1)
        wait_gathers(set0)

        @pl.when(b0 + 2 < nblk)
        def _i0():
            fire_idx(b0 + 2, set0)

        consume(b0, set0)

        @pl.when(b0 + 2 < nblk)
        def _g0():
            wait_idx(b0 + 2, set0)
            fire_gathers(set0)

        wait_gathers(set1)

        @pl.when(b1 + 2 < nblk)
        def _i1():
            fire_idx(b1 + 2, set1)

        consume(b1, set1)
        return carry

    lax.fori_loop(0, nblk // 2, body, 0)
    if nblk % 2:
        wait_gathers(set0)
        consume(nblk - 1, set0)


def _edge_gathers(s_packed, vf_packed, px, py, pz, src, tgt):
    n, dp = s_packed.shape
    e = src.shape[0]
    mesh = plsc.VectorSubcoreMesh(core_axis_name="c", subcore_axis_name="s")
    f32 = jnp.float32
    i32 = jnp.int32
    kfn = pl.kernel(
        _gather_body,
        mesh=mesh,
        out_type=(
            jax.ShapeDtypeStruct((e, dp), i32),
            jax.ShapeDtypeStruct((e, dp), i32),
            jax.ShapeDtypeStruct((e,), f32),
            jax.ShapeDtypeStruct((e,), f32),
            jax.ShapeDtypeStruct((e,), f32),
            jax.ShapeDtypeStruct((e,), f32),
        ),
        scratch_types=(
            [
                pltpu.VMEM((_GA_BLK,), jnp.int32),
                pltpu.VMEM((_GA_BLK,), jnp.int32),
                pltpu.VMEM((_GA_BLK, dp), i32),
                pltpu.VMEM((_GA_BLK, dp), i32),
            ]
            + [pltpu.VMEM((_GA_BLK,), f32) for _ in range(6)]
            + [pltpu.SemaphoreType.DMA, pltpu.SemaphoreType.DMA]
        ) * 2
        + [pltpu.VMEM((_GA_BLK,), f32) for _ in range(4)],
    )
    return kfn(s_packed, vf_packed, px, py, pz, src, tgt)


_TWO_PI = 6.2831853071795865
_COS_C = (
    0.9999994436787928,
    -0.49999558165605595,
    0.04166103279014615,
    -0.0013862747315839738,
    2.4253192495694853e-05,
    -2.2193949944515623e-07,
)


def _fast_cos(y):
    k = jnp.floor(y * (1.0 / _TWO_PI) + 0.5)
    t = y - k * _TWO_PI
    u = t * t
    r = jnp.float32(_COS_C[5])
    for c in _COS_C[4::-1]:
        r = r * u + jnp.float32(c)
    return r


def _unpack_halves(w):
    lo = jax.lax.bitcast_convert_type(jax.lax.shift_left(w, 16), jnp.float32)
    hi = jax.lax.bitcast_convert_type(
        jax.lax.bitwise_and(w, jnp.int32(-65536)), jnp.float32
    )
    return lo, hi


def _msg_body(ed_ref, ssrc_ref, vfsrc_ref, e32_ref, rw_ref, rb_ref, out_ref):
    d2 = ed_ref[:, 3:4]
    dist = jnp.sqrt(d2)
    inv_dist = 1.0 / dist
    ds_scaled = dist * (1.0 / _CUT)
    rbf = _fast_cos(ds_scaled * e32_ref[...] - (0.5 * jnp.pi)) * inv_dist
    filt = jnp.dot(rbf, rw_ref[...], preferred_element_type=jnp.float32)
    filt = filt + rb_ref[...]
    filt = jnp.where(
        filt < _CUT, 0.5 * (1.0 + _fast_cos(filt * (jnp.pi / _CUT))), 0.0
    )
    d = ssrc_ref.shape[1] // 2
    s_lo, s_hi = _unpack_halves(ssrc_ref[...])
    v_lo, v_hi = _unpack_halves(vfsrc_ref[...])
    scalar = s_lo[:, :d] * filt[:, :d]
    equiv = s_lo[:, d:] * filt[:, d:2 * d]
    invar = s_hi[:, :d] * filt[:, 2 * d:]
    out_ref[0] = scalar
    vparts = (v_lo[:, :d], v_lo[:, d:], v_hi[:, :d])
    for k in range(3):
        ru_k = ed_ref[:, k:k + 1] * inv_dist
        out_ref[k + 1] = invar * ru_k + equiv * vparts[k]


def _edge_messages(ed_t, s_src, vf_src, e32, rw, rb):
    e, dp = s_src.shape
    d = dp // 2
    d3 = 3 * d
    blk = 2560
    grid = e // blk
    return pl.pallas_call(
        _msg_body,
        grid=(grid,),
        in_specs=[
            pl.BlockSpec((blk, 4), lambda i: (i, 0)),
            pl.BlockSpec((blk, dp), lambda i: (i, 0)),
            pl.BlockSpec((blk, dp), lambda i: (i, 0)),
            pl.BlockSpec((1, e32.shape[1]), lambda i: (0, 0)),
            pl.BlockSpec((e32.shape[1], d3), lambda i: (0, 0)),
            pl.BlockSpec((1, d3), lambda i: (0, 0)),
        ],
        out_specs=pl.BlockSpec((4, blk, d), lambda i: (0, i, 0)),
        out_shape=jax.ShapeDtypeStruct((4, e, d), jnp.float32),
    )(ed_t, s_src, vf_src, e32, rw, rb)


_SC_BLK = 80


def _scatter_body(msg_h, tgt_h, zeros_h, out_h, acc,
                  idx0, buf0, sem0, idx1, buf1, sem1):
    n_edges = tgt_h.shape[0]
    n_nodes = zeros_h.shape[0]
    per_tile_e = n_edges // _NS
    rows_per_tile = (n_nodes // _NS) // 8 * 8
    rem = n_nodes - rows_per_tile * _NS
    nblk = per_tile_e // _SC_BLK
    c = lax.axis_index("c")
    s = lax.axis_index("s")
    r0 = s * rows_per_tile
    rrem = rows_per_tile * _NS
    ebase = s * per_tile_e

    for j in range(2):
        chunk = c * 2 + j
        pltpu.sync_copy(
            zeros_h.at[pl.ds(r0, rows_per_tile), :],
            acc.at[pl.ds(r0, rows_per_tile), :],
        )
        if rem:
            @pl.when(s == _NS - 1)
            def _zero_rem():
                pltpu.sync_copy(
                    zeros_h.at[pl.ds(rrem, rem), :],
                    acc.at[pl.ds(rrem, rem), :],
                )
        plsc.subcore_barrier()

        def start(b, idx_v, buf, sem):
            base = ebase + b * _SC_BLK
            pltpu.async_copy(tgt_h.at[pl.ds(base, _SC_BLK)], idx_v, sem)
            pltpu.async_copy(msg_h.at[chunk, pl.ds(base, _SC_BLK), :], buf, sem)

        def finish(b, idx_v, buf, sem):
            base = ebase + b * _SC_BLK
            pltpu.make_async_copy(
                tgt_h.at[pl.ds(base, _SC_BLK)], idx_v, sem
            ).wait()
            pltpu.make_async_copy(
                msg_h.at[chunk, pl.ds(base, _SC_BLK), :], buf, sem
            ).wait()
            pltpu.sync_copy(buf, acc.at[idx_v], add=True)

        start(0, idx0, buf0, sem0)

        def body(g, carry):
            b0 = g * 2
            start(b0 + 1, idx1, buf1, sem1)
            finish(b0, idx0, buf0, sem0)

            @pl.when(b0 + 2 < nblk)
            def _next():
                start(b0 + 2, idx0, buf0, sem0)

            finish(b0 + 1, idx1, buf1, sem1)
            return carry

        lax.fori_loop(0, nblk // 2, body, 0)
        plsc.subcore_barrier()
        pltpu.sync_copy(
            acc.at[pl.ds(r0, rows_per_tile), :],
            out_h.at[chunk, pl.ds(r0, rows_per_tile), :],
        )
        if rem:
            @pl.when(s == _NS - 1)
            def _drain_rem():
                pltpu.sync_copy(
                    acc.at[pl.ds(rrem, rem), :],
                    out_h.at[chunk, pl.ds(rrem, rem), :],
                )


def _scatter_add(msg4, tgt, zeros):
    e = tgt.shape[0]
    n, d = zeros.shape
    mesh = plsc.VectorSubcoreMesh(core_axis_name="c", subcore_axis_name="s")
    kfn = pl.kernel(
        _scatter_body,
        mesh=mesh,
        out_type=jax.ShapeDtypeStruct((4, n, d), jnp.float32),
        scratch_types=[
            pltpu.VMEM_SHARED((n, d), jnp.float32),
            pltpu.VMEM((_SC_BLK,), jnp.int32),
            pltpu.VMEM((_SC_BLK, d), jnp.float32),
            pltpu.SemaphoreType.DMA,
            pltpu.VMEM((_SC_BLK,), jnp.int32),
            pltpu.VMEM((_SC_BLK, d), jnp.float32),
            pltpu.SemaphoreType.DMA,
        ],
    )
    return kfn(msg4, tgt, zeros)


def kernel(vectorial_feat, scalar_feat, node_pos, edge_index, W1, b1, W2, b2,
           rbf_W, rbf_b, expanded_distance):
    n, d = scalar_feat.shape
    e = edge_index.shape[1]
    r = rbf_W.shape[0]
    d3 = 3 * d

    perm = (jnp.arange(d3) % d) * 3 + (jnp.arange(d3) // d)
    rwp = rbf_W[:, perm]
    rbp = rbf_b[perm].reshape(1, d3)
    d3e = 4 * d
    dp = d3e // 2
    w2p = jnp.concatenate([W2[:, perm], jnp.zeros((d, d), jnp.float32)], axis=1)
    b2p = jnp.concatenate([b2[perm], jnp.zeros((d,), jnp.float32)]).reshape(1, d3e)

    rp = 32
    rwp = jnp.concatenate([rwp, jnp.zeros((rp - r, d3), jnp.float32)], axis=0)
    e32 = jnp.concatenate(
        [expanded_distance, jnp.zeros((rp - r,), jnp.float32)]
    ).reshape(1, rp)

    vf_t = vectorial_feat.transpose(0, 2, 1).reshape(n, d3)
    rnd = jnp.int32(0x8000)
    vf_lo = jax.lax.bitcast_convert_type(vf_t[:, :dp], jnp.int32) + rnd
    vf_hi_f = jnp.concatenate(
        [vf_t[:, dp:], jnp.zeros((n, d), jnp.float32)], axis=1
    )
    vf_hi = jax.lax.bitcast_convert_type(vf_hi_f, jnp.int32) + rnd
    vf_packed = jax.lax.bitwise_or(
        jax.lax.shift_right_logical(vf_lo, 16),
        jax.lax.bitwise_and(vf_hi, jnp.int32(-65536)),
    )
    src = edge_index[0]
    tgt = edge_index[1]
    px = node_pos[:, 0]
    py = node_pos[:, 1]
    pz = node_pos[:, 2]

    s_packed = _node_mlp(scalar_feat, W1, b1.reshape(1, d), w2p, b2p)
    s_src, vf_src, ex, ey, ez, ed2 = _edge_gathers(
        s_packed, vf_packed, px, py, pz, src, tgt
    )
    ed_t = jnp.stack([ex, ey, ez, ed2], axis=1)
    msg4 = _edge_messages(ed_t, s_src, vf_src, e32, rwp, rbp)
    out4 = _scatter_add(msg4, tgt, jnp.zeros((n, d), jnp.float32))

    scalar_out = out4[0]
    vec_out = jnp.transpose(out4[1:], (1, 2, 0))
    return (vec_out, scalar_out)
